# Initial kernel scaffold; baseline (speedup 1.0000x reference)
import jax
import jax.numpy as jnp
from jax.experimental import pallas as pl


def kernel(input, size, k):
    raise NotImplementedError("write your pallas kernel here")


# speedup vs baseline: 552.1900x; 552.1900x over previous
import functools

import jax
import jax.numpy as jnp
from jax import lax
from jax.experimental import pallas as pl
from jax.experimental.pallas import tpu as pltpu
from jax.experimental.pallas import tpu_sc as plsc

B = 128
SEG = 32768
K = 10
L = 16
NC = 2
NS = 16
NW = NC * NS
SEG_PER_W = B // NW
ROWS = SEG // L
CHUNK_ROWS = 16
NCHUNK = ROWS // CHUNK_ROWS
CHUNK = CHUNK_ROWS * L

_MESH = plsc.VectorSubcoreMesh(core_axis_name="c", subcore_axis_name="s")


def _gat(x, idx):
    return lax.gather(
        x, idx.reshape(L, 1),
        lax.GatherDimensionNumbers(
            offset_dims=(), collapsed_slice_dims=(0,), start_index_map=(0,)),
        (1,), unique_indices=False, indices_are_sorted=False,
        mode=lax.GatherScatterMode.PROMISE_IN_BOUNDS)


def _allmax(x, iot):
    for sh in (8, 4, 2, 1):
        x = jnp.maximum(x, _gat(x, iot ^ sh))
    return x


def _allmin(x, iot):
    for sh in (8, 4, 2, 1):
        x = jnp.minimum(x, _gat(x, iot ^ sh))
    return x


@functools.partial(
    pl.kernel,
    out_type=jax.ShapeDtypeStruct((NW * 2 * L,), jnp.float32),
    mesh=_MESH,
    scratch_types=[
        pltpu.VMEM((SEG,), jnp.float32),
        pltpu.VMEM((2 * L,), jnp.float32),
    ],
)
def _stats_kernel(x_hbm, out_hbm, buf, st):
    wid = lax.axis_index("s") * NC + lax.axis_index("c")
    mx = jnp.full((L,), -jnp.inf, jnp.float32)
    mn = jnp.full((L,), jnp.inf, jnp.float32)
    for s in range(SEG_PER_W):
        seg = wid * SEG_PER_W + s
        pltpu.sync_copy(x_hbm.at[pl.ds(seg * SEG, SEG)], buf)

        def body(i, carry):
            cmx, cmn = carry
            base = i * (8 * L)
            for r in range(8):
                v = buf[pl.ds(base + r * L, L)]
                cmx = jnp.maximum(cmx, v)
                cmn = jnp.minimum(cmn, v)
            return cmx, cmn

        mx, mn = lax.fori_loop(0, ROWS // 8, body, (mx, mn))
    st[pl.ds(0, L)] = mx
    st[pl.ds(L, L)] = mn
    pltpu.sync_copy(st, out_hbm.at[pl.ds(wid * 2 * L, 2 * L)])


@functools.partial(
    pl.kernel,
    out_type=(
        jax.ShapeDtypeStruct((B * L,), jnp.float32),
        jax.ShapeDtypeStruct((B * L,), jnp.int32),
    ),
    mesh=_MESH,
    scratch_types=[
        pltpu.VMEM((SEG,), jnp.float32),
        pltpu.VMEM((NCHUNK * L,), jnp.float32),
        pltpu.VMEM((NCHUNK * L,), jnp.int32),
        pltpu.VMEM((NW * 2 * L,), jnp.float32),
        pltpu.VMEM((SEG_PER_W * L,), jnp.float32),
        pltpu.VMEM((SEG_PER_W * L,), jnp.int32),
    ],
)
def _topk_kernel(x_hbm, stats_hbm, outv_hbm, outi_hbm,
                 buf, mref, rref, stats, outv, outi):
    wid = lax.axis_index("s") * NC + lax.axis_index("c")
    iot = lax.iota(jnp.int32, L)

    pltpu.sync_copy(stats_hbm, stats)

    def sbody(i, carry):
        gmx, gmn = carry
        gmx = jnp.maximum(gmx, stats[pl.ds(i * 2 * L, L)])
        gmn = jnp.minimum(gmn, stats[pl.ds(i * 2 * L + L, L)])
        return gmx, gmn

    gmx, gmn = lax.fori_loop(
        0, NW, sbody,
        (jnp.full((L,), -jnp.inf, jnp.float32),
         jnp.full((L,), jnp.inf, jnp.float32)))
    mxv = _allmax(gmx, iot)
    mnv = _allmin(gmn, iot)
    neg_off = -((mxv - mnv) * jnp.float32(4.0))

    for s in range(SEG_PER_W):
        seg = wid * SEG_PER_W + s
        t = neg_off * seg.astype(jnp.float32)
        pltpu.sync_copy(x_hbm.at[pl.ds(seg * SEG, SEG)], buf)

        def abody(c, _):
            m = jnp.full((L,), -jnp.inf, jnp.float32)
            rr = jnp.zeros((L,), jnp.int32)
            base = c * CHUNK
            for r in range(CHUNK_ROWS):
                key = buf[pl.ds(base + r * L, L)] + t
                upd = key > m
                m = jnp.where(upd, key, m)
                rr = jnp.where(upd, jnp.int32(r), rr)
            mref[pl.ds(c * L, L)] = m
            rref[pl.ds(c * L, L)] = rr
            return 0

        lax.fori_loop(0, NCHUNK, abody, 0)

        def ebody(kk, _):
            def scan(c, carry):
                bm, bc, br = carry
                m = mref[pl.ds(c * L, L)]
                rr = rref[pl.ds(c * L, L)]
                upd = m > bm
                bm = jnp.where(upd, m, bm)
                bc = jnp.where(upd, c, bc)
                br = jnp.where(upd, rr, br)
                return bm, bc, br

            bm, bc, br = lax.fori_loop(
                0, NCHUNK, scan,
                (jnp.full((L,), -jnp.inf, jnp.float32),
                 jnp.zeros((L,), jnp.int32),
                 jnp.zeros((L,), jnp.int32)))
            vmax = _allmax(bm, iot)
            gidx = bc * CHUNK + br * L + iot
            cand = jnp.where(bm == vmax, gidx, jnp.int32(2**30))
            gv = _allmin(cand, iot)
            g = gv[0]

            rowbase = (g // L) * L
            lane = g - rowbase
            vv = buf[pl.ds(rowbase, L)]
            selm = iot == lane
            valv = _gat(vv, jnp.full((L,), lane, jnp.int32))
            buf[pl.ds(rowbase, L)] = jnp.where(
                selm, jnp.float32(-jnp.inf), vv)

            selk = iot == kk
            outv[pl.ds(s * L, L)] = jnp.where(selk, valv, outv[pl.ds(s * L, L)])
            outi[pl.ds(s * L, L)] = jnp.where(selk, gv, outi[pl.ds(s * L, L)])

            c2 = g // CHUNK
            cbase = c2 * CHUNK
            m = jnp.full((L,), -jnp.inf, jnp.float32)
            rr = jnp.zeros((L,), jnp.int32)
            for r in range(CHUNK_ROWS):
                key = buf[pl.ds(cbase + r * L, L)] + t
                upd = key > m
                m = jnp.where(upd, key, m)
                rr = jnp.where(upd, jnp.int32(r), rr)
            mref[pl.ds(c2 * L, L)] = m
            rref[pl.ds(c2 * L, L)] = rr
            return 0

        lax.fori_loop(0, K, ebody, 0)

    pltpu.sync_copy(outv, outv_hbm.at[pl.ds(wid * SEG_PER_W * L, SEG_PER_W * L)])
    pltpu.sync_copy(outi, outi_hbm.at[pl.ds(wid * SEG_PER_W * L, SEG_PER_W * L)])


def kernel(input, size, k):
    del size, k
    stats = _stats_kernel(input)
    v, i = _topk_kernel(input, stats)
    value = v.reshape(B, L)[:, :K]
    index = i.reshape(B, L)[:, :K]
    return value, index


# speedup vs baseline: 813.0922x; 1.4725x over previous
import functools

import jax
import jax.numpy as jnp
from jax import lax
from jax.experimental import pallas as pl
from jax.experimental.pallas import tpu as pltpu
from jax.experimental.pallas import tpu_sc as plsc

B = 128
SEG = 32768
K = 10
L = 16
NC = 2
NS = 16
NW = NC * NS
SEG_PER_W = B // NW
ROWS = SEG // L
CHUNK_ROWS = 16
NCHUNK = ROWS // CHUNK_ROWS
CHUNK = CHUNK_ROWS * L
GROUP = 16
NGROUP = NCHUNK // GROUP
BIG = 2**30

_MESH = plsc.VectorSubcoreMesh(core_axis_name="c", subcore_axis_name="s")


def _gat(x, idx):
    return lax.gather(
        x, idx.reshape(L, 1),
        lax.GatherDimensionNumbers(
            offset_dims=(), collapsed_slice_dims=(0,), start_index_map=(0,)),
        (1,), unique_indices=False, indices_are_sorted=False,
        mode=lax.GatherScatterMode.PROMISE_IN_BOUNDS)


def _allmax(x, iot):
    for sh in (8, 4, 2, 1):
        x = jnp.maximum(x, _gat(x, iot ^ sh))
    return x


def _allmin(x, iot):
    for sh in (8, 4, 2, 1):
        x = jnp.minimum(x, _gat(x, iot ^ sh))
    return x


@functools.partial(
    pl.kernel,
    out_type=jax.ShapeDtypeStruct((NW * 2 * L,), jnp.float32),
    mesh=_MESH,
    scratch_types=[
        pltpu.VMEM((SEG,), jnp.float32),
        pltpu.VMEM((SEG,), jnp.float32),
        pltpu.VMEM((2 * L,), jnp.float32),
        pltpu.SemaphoreType.DMA,
        pltpu.SemaphoreType.DMA,
    ],
)
def _stats_kernel(x_hbm, out_hbm, buf0, buf1, st, sem0, sem1):
    wid = lax.axis_index("s") * NC + lax.axis_index("c")
    seg0 = wid * SEG_PER_W
    bufs = (buf0, buf1)
    sems = (sem0, sem1)
    mx = jnp.full((L,), -jnp.inf, jnp.float32)
    mn = jnp.full((L,), jnp.inf, jnp.float32)
    pend = pltpu.async_copy(x_hbm.at[pl.ds(seg0 * SEG, SEG)], buf0, sem0)
    for s in range(SEG_PER_W):
        buf = bufs[s % 2]
        pend.wait()
        if s + 1 < SEG_PER_W:
            pend = pltpu.async_copy(
                x_hbm.at[pl.ds((seg0 + s + 1) * SEG, SEG)],
                bufs[(s + 1) % 2], sems[(s + 1) % 2])

        def body(i, carry, buf=buf):
            cmx, cmn = carry
            base = i * (16 * L)
            for r in range(16):
                v = buf[pl.ds(base + r * L, L)]
                cmx = jnp.maximum(cmx, v)
                cmn = jnp.minimum(cmn, v)
            return cmx, cmn

        mx, mn = lax.fori_loop(0, ROWS // 16, body, (mx, mn))
    st[pl.ds(0, L)] = mx
    st[pl.ds(L, L)] = mn
    pltpu.sync_copy(st, out_hbm.at[pl.ds(wid * 2 * L, 2 * L)])


@functools.partial(
    pl.kernel,
    out_type=(
        jax.ShapeDtypeStruct((B * L,), jnp.float32),
        jax.ShapeDtypeStruct((B * L,), jnp.int32),
    ),
    mesh=_MESH,
    scratch_types=[
        pltpu.VMEM((SEG,), jnp.float32),
        pltpu.VMEM((SEG,), jnp.float32),
        pltpu.VMEM((NCHUNK * L,), jnp.float32),
        pltpu.VMEM((NCHUNK * L,), jnp.int32),
        pltpu.VMEM((NGROUP * L,), jnp.float32),
        pltpu.VMEM((NW * 2 * L,), jnp.float32),
        pltpu.VMEM((SEG_PER_W * L,), jnp.float32),
        pltpu.VMEM((SEG_PER_W * L,), jnp.int32),
        pltpu.SemaphoreType.DMA,
        pltpu.SemaphoreType.DMA,
    ],
)
def _topk_kernel(x_hbm, stats_hbm, outv_hbm, outi_hbm,
                 buf0, buf1, mref, rref, g2v, stats, outv, outi, sem0, sem1):
    wid = lax.axis_index("s") * NC + lax.axis_index("c")
    seg0 = wid * SEG_PER_W
    bufs = (buf0, buf1)
    sems = (sem0, sem1)
    iot = lax.iota(jnp.int32, L)

    pend = pltpu.async_copy(x_hbm.at[pl.ds(seg0 * SEG, SEG)], buf0, sem0)

    pltpu.sync_copy(stats_hbm, stats)

    def sbody(i, carry):
        gmx, gmn = carry
        gmx = jnp.maximum(gmx, stats[pl.ds(i * 2 * L, L)])
        gmn = jnp.minimum(gmn, stats[pl.ds(i * 2 * L + L, L)])
        return gmx, gmn

    gmx, gmn = lax.fori_loop(
        0, NW, sbody,
        (jnp.full((L,), -jnp.inf, jnp.float32),
         jnp.full((L,), jnp.inf, jnp.float32)))
    mxv = _allmax(gmx, iot)
    mnv = _allmin(gmn, iot)
    neg_off = -((mxv - mnv) * jnp.float32(4.0))

    for s in range(SEG_PER_W):
        buf = bufs[s % 2]
        seg = seg0 + s
        t = neg_off * seg.astype(jnp.float32)
        pend.wait()
        if s + 1 < SEG_PER_W:
            pend = pltpu.async_copy(
                x_hbm.at[pl.ds((seg + 1) * SEG, SEG)],
                bufs[(s + 1) % 2], sems[(s + 1) % 2])

        def abody(c, _, buf=buf, t=t):
            m = jnp.full((L,), -jnp.inf, jnp.float32)
            rr = jnp.zeros((L,), jnp.int32)
            base = c * CHUNK
            for r in range(CHUNK_ROWS):
                key = buf[pl.ds(base + r * L, L)] + t
                upd = key > m
                m = jnp.where(upd, key, m)
                rr = jnp.where(upd, jnp.int32(r), rr)
            mref[pl.ds(c * L, L)] = m
            rref[pl.ds(c * L, L)] = rr
            return 0

        lax.fori_loop(0, NCHUNK, abody, 0, unroll=2)

        def gbody(gg, _):
            bv = jnp.full((L,), -jnp.inf, jnp.float32)
            for cc in range(GROUP):
                bv = jnp.maximum(bv, mref[pl.ds((gg * GROUP + cc) * L, L)])
            g2v[pl.ds(gg * L, L)] = bv
            return 0

        lax.fori_loop(0, NGROUP, gbody, 0)

        def ebody(kk, _, buf=buf, t=t):
            bv = jnp.full((L,), -jnp.inf, jnp.float32)
            bg = jnp.zeros((L,), jnp.int32)
            for gg in range(NGROUP):
                val = g2v[pl.ds(gg * L, L)]
                upd = val > bv
                bv = jnp.where(upd, val, bv)
                bg = jnp.where(upd, jnp.int32(gg), bg)
            vmax = _allmax(bv, iot)
            gstar = _allmin(jnp.where(bv == vmax, bg, BIG), iot)[0]
            cbase0 = gstar * GROUP

            bm = jnp.full((L,), -jnp.inf, jnp.float32)
            bc = jnp.zeros((L,), jnp.int32)
            br = jnp.zeros((L,), jnp.int32)
            for cc in range(GROUP):
                m = mref[pl.ds((cbase0 + cc) * L, L)]
                rr = rref[pl.ds((cbase0 + cc) * L, L)]
                upd = m > bm
                bm = jnp.where(upd, m, bm)
                bc = jnp.where(upd, jnp.int32(cc), bc)
                br = jnp.where(upd, rr, br)
            gidx = (bc + cbase0) * CHUNK + br * L + iot
            gv = _allmin(jnp.where(bm == vmax, gidx, BIG), iot)
            g = gv[0]

            rowbase = (g // L) * L
            lane = g - rowbase
            vv = buf[pl.ds(rowbase, L)]
            selm = iot == lane
            valv = _gat(vv, jnp.full((L,), lane, jnp.int32))
            buf[pl.ds(rowbase, L)] = jnp.where(
                selm, jnp.float32(-jnp.inf), vv)

            selk = iot == kk
            outv[pl.ds(s * L, L)] = jnp.where(selk, valv, outv[pl.ds(s * L, L)])
            outi[pl.ds(s * L, L)] = jnp.where(selk, gv, outi[pl.ds(s * L, L)])

            c2 = g // CHUNK
            cbase = c2 * CHUNK
            m = jnp.full((L,), -jnp.inf, jnp.float32)
            rr = jnp.zeros((L,), jnp.int32)
            for r in range(CHUNK_ROWS):
                key = buf[pl.ds(cbase + r * L, L)] + t
                upd = key > m
                m = jnp.where(upd, key, m)
                rr = jnp.where(upd, jnp.int32(r), rr)
            mref[pl.ds(c2 * L, L)] = m
            rref[pl.ds(c2 * L, L)] = rr
            bv2 = jnp.full((L,), -jnp.inf, jnp.float32)
            for cc in range(GROUP):
                bv2 = jnp.maximum(bv2, mref[pl.ds((cbase0 + cc) * L, L)])
            g2v[pl.ds(gstar * L, L)] = bv2
            return 0

        lax.fori_loop(0, K, ebody, 0)

    pltpu.sync_copy(outv, outv_hbm.at[pl.ds(wid * SEG_PER_W * L, SEG_PER_W * L)])
    pltpu.sync_copy(outi, outi_hbm.at[pl.ds(wid * SEG_PER_W * L, SEG_PER_W * L)])


def kernel(input, size, k):
    del size, k
    stats = _stats_kernel(input)
    v, i = _topk_kernel(input, stats)
    value = v.reshape(B, L)[:, :K]
    index = i.reshape(B, L)[:, :K]
    return value, index


# speedup vs baseline: 858.8268x; 1.0562x over previous
import functools

import jax
import jax.numpy as jnp
from jax import lax
from jax.experimental import pallas as pl
from jax.experimental.pallas import tpu as pltpu
from jax.experimental.pallas import tpu_sc as plsc

B = 128
SEG = 32768
K = 10
L = 16
NC = 2
NS = 16
NW = NC * NS
SEG_PER_W = B // NW
ROWS = SEG // L
CHUNK_ROWS = 16
NCHUNK = ROWS // CHUNK_ROWS
CHUNK = CHUNK_ROWS * L
GROUP = 16
NGROUP = NCHUNK // GROUP
BIG = 2**30

_MESH = plsc.VectorSubcoreMesh(core_axis_name="c", subcore_axis_name="s")


def _gat(x, idx):
    return lax.gather(
        x, idx.reshape(L, 1),
        lax.GatherDimensionNumbers(
            offset_dims=(), collapsed_slice_dims=(0,), start_index_map=(0,)),
        (1,), unique_indices=False, indices_are_sorted=False,
        mode=lax.GatherScatterMode.PROMISE_IN_BOUNDS)


def _allmax(x, iot):
    for sh in (8, 4, 2, 1):
        x = jnp.maximum(x, _gat(x, iot ^ sh))
    return x


def _allmin(x, iot):
    for sh in (8, 4, 2, 1):
        x = jnp.minimum(x, _gat(x, iot ^ sh))
    return x


@functools.partial(
    pl.kernel,
    out_type=jax.ShapeDtypeStruct((NW * 2 * L,), jnp.float32),
    mesh=_MESH,
    scratch_types=[
        pltpu.VMEM((SEG,), jnp.float32),
        pltpu.VMEM((SEG,), jnp.float32),
        pltpu.VMEM((2 * L,), jnp.float32),
        pltpu.SemaphoreType.DMA,
        pltpu.SemaphoreType.DMA,
    ],
)
def _stats_kernel(x_hbm, out_hbm, buf0, buf1, st, sem0, sem1):
    wid = lax.axis_index("s") * NC + lax.axis_index("c")
    seg0 = wid * SEG_PER_W
    bufs = (buf0, buf1)
    sems = (sem0, sem1)
    mx = jnp.full((L,), -jnp.inf, jnp.float32)
    mn = jnp.full((L,), jnp.inf, jnp.float32)
    pend = pltpu.async_copy(x_hbm.at[pl.ds(seg0 * SEG, SEG)], buf0, sem0)
    for s in range(SEG_PER_W):
        buf = bufs[s % 2]
        pend.wait()
        if s + 1 < SEG_PER_W:
            pend = pltpu.async_copy(
                x_hbm.at[pl.ds((seg0 + s + 1) * SEG, SEG)],
                bufs[(s + 1) % 2], sems[(s + 1) % 2])

        def body(i, carry, buf=buf):
            acc = list(carry)
            base = i * (16 * L)
            for r in range(16):
                v = buf[pl.ds(base + r * L, L)]
                acc[r % 4] = jnp.maximum(acc[r % 4], v)
                acc[4 + r % 4] = jnp.minimum(acc[4 + r % 4], v)
            return tuple(acc)

        accs = lax.fori_loop(
            0, ROWS // 16, body,
            tuple([mx] * 4 + [mn] * 4))
        mx = jnp.maximum(jnp.maximum(accs[0], accs[1]),
                         jnp.maximum(accs[2], accs[3]))
        mn = jnp.minimum(jnp.minimum(accs[4], accs[5]),
                         jnp.minimum(accs[6], accs[7]))
    st[pl.ds(0, L)] = mx
    st[pl.ds(L, L)] = mn
    pltpu.sync_copy(st, out_hbm.at[pl.ds(wid * 2 * L, 2 * L)])


@functools.partial(
    pl.kernel,
    out_type=(
        jax.ShapeDtypeStruct((B * L,), jnp.float32),
        jax.ShapeDtypeStruct((B * L,), jnp.int32),
    ),
    mesh=_MESH,
    scratch_types=[
        pltpu.VMEM((SEG,), jnp.float32),
        pltpu.VMEM((SEG,), jnp.float32),
        pltpu.VMEM((NCHUNK * L,), jnp.float32),
        pltpu.VMEM((NGROUP * L,), jnp.float32),
        pltpu.VMEM((NW * 2 * L,), jnp.float32),
        pltpu.VMEM((SEG_PER_W * L,), jnp.float32),
        pltpu.VMEM((SEG_PER_W * L,), jnp.int32),
        pltpu.SemaphoreType.DMA,
        pltpu.SemaphoreType.DMA,
    ],
)
def _topk_kernel(x_hbm, stats_hbm, outv_hbm, outi_hbm,
                 buf0, buf1, mref, g2v, stats, outv, outi, sem0, sem1):
    wid = lax.axis_index("s") * NC + lax.axis_index("c")
    seg0 = wid * SEG_PER_W
    bufs = (buf0, buf1)
    sems = (sem0, sem1)
    iot = lax.iota(jnp.int32, L)

    pend = pltpu.async_copy(x_hbm.at[pl.ds(seg0 * SEG, SEG)], buf0, sem0)

    pltpu.sync_copy(stats_hbm, stats)

    def sbody(i, carry):
        gmx, gmn = carry
        gmx = jnp.maximum(gmx, stats[pl.ds(i * 2 * L, L)])
        gmn = jnp.minimum(gmn, stats[pl.ds(i * 2 * L + L, L)])
        return gmx, gmn

    gmx, gmn = lax.fori_loop(
        0, NW, sbody,
        (jnp.full((L,), -jnp.inf, jnp.float32),
         jnp.full((L,), jnp.inf, jnp.float32)))
    mxv = _allmax(gmx, iot)
    mnv = _allmin(gmn, iot)
    neg_off = -((mxv - mnv) * jnp.float32(4.0))

    for s in range(SEG_PER_W):
        buf = bufs[s % 2]
        seg = seg0 + s
        t = neg_off * seg.astype(jnp.float32)
        pend.wait()
        if s + 1 < SEG_PER_W:
            pend = pltpu.async_copy(
                x_hbm.at[pl.ds((seg + 1) * SEG, SEG)],
                bufs[(s + 1) % 2], sems[(s + 1) % 2])

        def abody(c, _, buf=buf, t=t):
            acc = [jnp.full((L,), -jnp.inf, jnp.float32) for _ in range(4)]
            base = c * CHUNK
            for r in range(CHUNK_ROWS):
                acc[r % 4] = jnp.maximum(acc[r % 4], buf[pl.ds(base + r * L, L)])
            m = jnp.maximum(jnp.maximum(acc[0], acc[1]),
                            jnp.maximum(acc[2], acc[3]))
            mref[pl.ds(c * L, L)] = m + t
            return 0

        lax.fori_loop(0, NCHUNK, abody, 0, unroll=2)

        def gbody(gg, _):
            bv = jnp.full((L,), -jnp.inf, jnp.float32)
            for cc in range(GROUP):
                bv = jnp.maximum(bv, mref[pl.ds((gg * GROUP + cc) * L, L)])
            g2v[pl.ds(gg * L, L)] = bv
            return 0

        lax.fori_loop(0, NGROUP, gbody, 0)

        def ebody(kk, _, buf=buf, t=t):
            bv = jnp.full((L,), -jnp.inf, jnp.float32)
            bg = jnp.zeros((L,), jnp.int32)
            for gg in range(NGROUP):
                val = g2v[pl.ds(gg * L, L)]
                upd = val > bv
                bv = jnp.where(upd, val, bv)
                bg = jnp.where(upd, jnp.int32(gg), bg)
            vmax = _allmax(bv, iot)
            gstar = _allmin(jnp.where(bv == vmax, bg, BIG), iot)[0]
            cbase0 = gstar * GROUP

            bm = jnp.full((L,), -jnp.inf, jnp.float32)
            bc = jnp.zeros((L,), jnp.int32)
            for cc in range(GROUP):
                m = mref[pl.ds((cbase0 + cc) * L, L)]
                upd = m > bm
                bm = jnp.where(upd, m, bm)
                bc = jnp.where(upd, jnp.int32(cc), bc)
            cstar = _allmin(jnp.where(bm == vmax, bc + cbase0, BIG), iot)[0]
            cbase = cstar * CHUNK

            rbest = jnp.full((L,), CHUNK_ROWS, jnp.int32)
            for r in range(CHUNK_ROWS - 1, -1, -1):
                key = buf[pl.ds(cbase + r * L, L)] + t
                rbest = jnp.where(key == vmax, jnp.int32(r), rbest)
            gidx = cbase + rbest * L + iot
            gv = _allmin(
                jnp.where(rbest < CHUNK_ROWS, gidx, BIG), iot)
            g = gv[0]

            rowbase = (g // L) * L
            lane = g - rowbase
            vv = buf[pl.ds(rowbase, L)]
            selm = iot == lane
            valv = _gat(vv, jnp.full((L,), lane, jnp.int32))
            buf[pl.ds(rowbase, L)] = jnp.where(
                selm, jnp.float32(-jnp.inf), vv)

            selk = iot == kk
            outv[pl.ds(s * L, L)] = jnp.where(selk, valv, outv[pl.ds(s * L, L)])
            outi[pl.ds(s * L, L)] = jnp.where(selk, gv, outi[pl.ds(s * L, L)])

            acc = [jnp.full((L,), -jnp.inf, jnp.float32) for _ in range(4)]
            for r in range(CHUNK_ROWS):
                acc[r % 4] = jnp.maximum(acc[r % 4], buf[pl.ds(cbase + r * L, L)])
            m2 = jnp.maximum(jnp.maximum(acc[0], acc[1]),
                             jnp.maximum(acc[2], acc[3]))
            mref[pl.ds(cstar * L, L)] = m2 + t
            bv2 = jnp.full((L,), -jnp.inf, jnp.float32)
            for cc in range(GROUP):
                bv2 = jnp.maximum(bv2, mref[pl.ds((cbase0 + cc) * L, L)])
            g2v[pl.ds(gstar * L, L)] = bv2
            return 0

        lax.fori_loop(0, K, ebody, 0)

    pltpu.sync_copy(outv, outv_hbm.at[pl.ds(wid * SEG_PER_W * L, SEG_PER_W * L)])
    pltpu.sync_copy(outi, outi_hbm.at[pl.ds(wid * SEG_PER_W * L, SEG_PER_W * L)])


def kernel(input, size, k):
    del size, k
    stats = _stats_kernel(input)
    v, i = _topk_kernel(input, stats)
    value = v.reshape(B, L)[:, :K]
    index = i.reshape(B, L)[:, :K]
    return value, index
